# baseline (speedup 1.0000x reference)
import jax
import jax.numpy as jnp
from jax import lax
from jax.experimental import pallas as pl
from jax.experimental.pallas import tpu as pltpu

B, T, N, H, C = 16, 64, 62, 2, 256
POS = 16
E = 4000
R = B * N

_f32 = jnp.float32


def _iota(shape, dim):
    return lax.broadcasted_iota(jnp.int32, shape, dim)


def _weights_body(w0_refs, b0_refs, w1_refs, b1_refs, w2_refs, b2_refs,
                  wsw_ref, wsb_ref, fc3w_ref, fc3b_ref, fc2w_ref, fc2b_ref,
                  fc1w_ref, fc1b_ref, gatb8_ref, edge_ref,
                  T_refs, bN_refs, Gp_refs, Gs_refs, bP_refs,
                  F_ref, bF_ref, W2m_ref, cv_ref, M_ref):
    def sel(n_in, k):
        return ((_iota((2 * n_in, n_in), 0) == _iota((2 * n_in, n_in), 1) * 2 + k)
                ).astype(_f32)

    SEL32 = [sel(32, 0), sel(32, 1)]
    SEL16 = [sel(16, 0), sel(16, 1)]

    wsw = wsw_ref[...]
    wsb = wsb_ref[...]
    d3 = _iota((64, 1, 64), 2) - _iota((64, 1, 64), 0)
    d4 = _iota((1, 16, 1, 16), 3) - _iota((1, 16, 1, 16), 1)
    wrow = _iota((1, 80), 1)

    def dotf(a, b, dn):
        return lax.dot_general(a, b, (dn, ((), ())),
                               preferred_element_type=_f32)

    for j in range(3):
        w0 = w0_refs[j][...]
        w1 = w1_refs[j][...]
        w2 = w2_refs[j][...]
        L1 = [jnp.dot(w0, SEL32[1], preferred_element_type=_f32),
              jnp.dot(w0, SEL32[0], preferred_element_type=_f32)]
        L2 = [jnp.dot(w1, SEL16[1], preferred_element_type=_f32),
              jnp.dot(w1, SEL16[0], preferred_element_type=_f32)]
        L3 = [jnp.dot(w2, SEL16[1], preferred_element_type=_f32),
              jnp.dot(w2, SEL16[0], preferred_element_type=_f32)]

        Tacc = jnp.zeros((64, 16, 64), _f32)
        Gp = jnp.zeros((32, 16, 16, 16), _f32)
        Gs = jnp.zeros((8, 16, 16), _f32)
        betaT = (b2_refs[j][...]
                 * jnp.ones((1, 80), _f32))
        for a in (0, 1):
            for b in (0, 1):
                colab = jnp.dot(L3[a], jnp.dot(L2[b], b0_refs[j][...],
                                               preferred_element_type=_f32),
                                preferred_element_type=_f32)
                betaT = betaT + colab * (wrow >= 4 * a + 2 * b).astype(_f32)
            cola = jnp.dot(L3[a], b1_refs[j][...],
                           preferred_element_type=_f32)
            betaT = betaT + cola * (wrow >= 4 * a).astype(_f32)

        for t in range(8):
            a, b, c = (t >> 2) & 1, (t >> 1) & 1, t & 1
            P21 = jnp.dot(L2[b], L1[c], preferred_element_type=_f32)
            A_t = jnp.dot(L3[a], P21, preferred_element_type=_f32)
            AT_t = dotf(P21, L3[a], ((0,), (1,)))
            v_t = jnp.dot(A_t, wsw, preferred_element_type=_f32)
            awsb = jnp.dot(A_t, wsb, preferred_element_type=_f32)
            betaT = betaT + awsb * ((wrow >= t) & (wrow < 64 + t)).astype(_f32)
            Tacc = Tacc + (d3 == t).astype(_f32) * v_t[None, :, :]
            Gp = Gp + (d4 == t).astype(_f32) * AT_t[:, None, :, None]
            if t >= 1:
                m3 = (_iota((8, 1, 16), 2) == _iota((8, 1, 16), 0) - 8 + t)
                Gs = Gs + m3.astype(_f32) * v_t[None, :, :]

        T_refs[j][...] = Tacc
        bN_refs[j][...] = betaT[:, :64]
        bP_refs[j][...] = betaT[:, 64:]
        Gp_refs[j][...] = Gp
        Gs_refs[j][...] = Gs

    fc3T = jnp.transpose(fc3w_ref[...])
    mwp = (_iota((1, 16, 1, 16), 1) == _iota((1, 16, 1, 16), 3)).astype(_f32)
    F_ref[...] = mwp * fc3T[:, None, :, None]
    bF_ref[...] = fc3b_ref[...] * jnp.ones((1, 16), _f32)

    fc2T = jnp.transpose(fc2w_ref[...])
    mtv = (_iota((1, 64, 64), 1) == _iota((1, 64, 64), 2)).astype(_f32)
    W2m_ref[...] = mtv * fc2T[:, :, None]
    gbrow = jnp.sum(gatb8_ref[...] * fc2T, axis=0, keepdims=True)
    w1s = jnp.sum(fc1w_ref[...], axis=1, keepdims=True)
    cv_ref[...] = fc1b_ref[...] + w1s * fc2b_ref[...] + w1s * gbrow

    srcrow = edge_ref[0:1, :]
    dstrow = edge_ref[1:2, :]
    D = (_iota((N, E), 0) == dstrow).astype(_f32)
    S = (_iota((N, E), 0) == srcrow).astype(_f32)
    eye = (_iota((N, N), 0) == _iota((N, N), 1)).astype(_f32)
    M_ref[...] = dotf(D, S, ((1,), (1,))) + eye


def _front_body(s_ref, p_ref, T1_ref, T2_ref, T3_ref, bN1_ref, bN2_ref,
                bN3_ref, Gp1_ref, Gp2_ref, Gp3_ref, Gs1_ref, Gs2_ref, Gs3_ref,
                bP1_ref, bP2_ref, bP3_ref, F_ref, bF_ref,
                wl_ref, bl_ref, wr_ref, br_ref,
                xl_ref, xr_ref, pos_ref):
    s = s_ref[...]
    stail = s[:, 56:64]
    p = p_ref[...]

    def node_branch(T_r, bN_r):
        return jnp.dot(s, T_r[...], preferred_element_type=_f32) + bN_r[...]

    def pos_branch(Gp_r, Gs_r, bP_r):
        return (jnp.dot(p, Gp_r[...], preferred_element_type=_f32)
                + jnp.dot(stail, Gs_r[...], preferred_element_type=_f32)
                + bP_r[...])

    def glu(o1, o2, o3):
        return jnp.maximum(jnp.tanh(o1) * jax.nn.sigmoid(o2) + o3, 0.0)

    g_node = glu(node_branch(T1_ref, bN1_ref),
                 node_branch(T2_ref, bN2_ref),
                 node_branch(T3_ref, bN3_ref))
    g_pos = glu(pos_branch(Gp1_ref, Gs1_ref, bP1_ref),
                pos_branch(Gp2_ref, Gs2_ref, bP2_ref),
                pos_branch(Gp3_ref, Gs3_ref, bP3_ref))

    xl_ref[...] = (jnp.dot(g_node, wl_ref[...], preferred_element_type=_f32)
                   + bl_ref[...]).reshape(B, N, H * C)
    xr_ref[...] = (jnp.dot(g_node, wr_ref[...], preferred_element_type=_f32)
                   + br_ref[...]).reshape(B, N, H * C)
    pos_ref[...] = jnp.dot(g_pos, F_ref[...],
                           preferred_element_type=_f32) + bF_ref[...]


def _gat_body(xl_ref, xr_ref, att_ref, M_ref, w1_ref, W2m_ref, cv_ref,
              pre_ref):
    M = M_ref[...]
    mask = M > 0.0
    Xl_all = xl_ref[0]
    Xr_all = xr_ref[0]
    qs = []
    for h in range(H):
        Xl = Xl_all[:, h * C:(h + 1) * C]
        Xr = Xr_all[:, h * C:(h + 1) * C]
        att = att_ref[h:h + 1, :]
        amag = 0.4 * jnp.abs(att)
        asgn = jnp.where(att >= 0.0, 1.0, -1.0)[None, :, :]
        Yl = Xl * amag
        Yr = Xr * amag
        al = 0.6 * lax.dot_general(att, Xl, (((1,), (1,)), ((), ())),
                                   preferred_element_type=_f32)
        ar = 0.6 * lax.dot_general(Xr, att, (((1,), (1,)), ((), ())),
                                   preferred_element_type=_f32)
        Z = Yl[None, :, :] + Yr[:, None, :]
        Sabs = jnp.sum(jnp.abs(Z) * asgn, axis=-1)
        S = Sabs + al + ar
        Sm = jnp.where(mask, S, -1e30)
        mx = jnp.max(Sm, axis=1, keepdims=True)
        P = jnp.exp(Sm - mx) * M
        den = jnp.sum(P, axis=1, keepdims=True)
        Aw = P / (den + 1e-16)
        tau = jnp.dot(w1_ref[...], Aw, preferred_element_type=_f32)
        qs.append(jnp.dot(tau, Xl, preferred_element_type=_f32))
    q = jnp.concatenate(qs, axis=1)
    acc = jnp.dot(q, W2m_ref[...], preferred_element_type=_f32)
    pre_ref[...] = jax.nn.sigmoid(acc + cv_ref[...])[None]


def kernel(data, position_embed, edge_index, ws_w, ws_b,
           d1w0, d1b0, d1w1, d1b1, d1w2, d1b2,
           d2w0, d2b0, d2w1, d2b1, d2w2, d2b2,
           d3w0, d3b0, d3w1, d3b1, d3w2, d3b2,
           fc3_w, fc3_b, gat_wl, gat_bl, gat_wr, gat_br, gat_att, gat_b,
           fc2_w, fc2_b, fc1_w, fc1_b):
    data = data.astype(_f32)
    train = data[:, :62, :]
    target = data[:, 62:63, :]
    s = train.reshape(R, T)
    p = jnp.transpose(position_embed, (0, 2, 1, 3)).reshape(R, 32 * POS)

    w0s = [d1w0.reshape(16, 64), d2w0.reshape(16, 64), d3w0.reshape(16, 64)]
    w1s = [d1w1.reshape(16, 32), d2w1.reshape(16, 32), d3w1.reshape(16, 32)]
    w2s = [d1w2.reshape(16, 32), d2w2.reshape(16, 32), d3w2.reshape(16, 32)]
    b0s = [d1b0[:, None], d2b0[:, None], d3b0[:, None]]
    b1s = [d1b1[:, None], d2b1[:, None], d3b1[:, None]]
    b2s = [d1b2[:, None], d2b2[:, None], d3b2[:, None]]

    wshape = [jax.ShapeDtypeStruct((64, 16, 64), _f32)] * 3 \
        + [jax.ShapeDtypeStruct((16, 64), _f32)] * 3 \
        + [jax.ShapeDtypeStruct((32, 16, 16, 16), _f32)] * 3 \
        + [jax.ShapeDtypeStruct((8, 16, 16), _f32)] * 3 \
        + [jax.ShapeDtypeStruct((16, 16), _f32)] * 3 \
        + [jax.ShapeDtypeStruct((16, 16, 32, 16), _f32),
           jax.ShapeDtypeStruct((32, 16), _f32),
           jax.ShapeDtypeStruct((8, 64, 64), _f32),
           jax.ShapeDtypeStruct((1, 64), _f32),
           jax.ShapeDtypeStruct((N, N), _f32)]

    def _wk(*refs):
        _weights_body(refs[0:3], refs[3:6], refs[6:9], refs[9:12],
                      refs[12:15], refs[15:18], refs[18], refs[19], refs[20],
                      refs[21], refs[22], refs[23], refs[24], refs[25],
                      refs[26], refs[27],
                      refs[28:31], refs[31:34], refs[34:37], refs[37:40],
                      refs[40:43], refs[43], refs[44], refs[45], refs[46],
                      refs[47])

    wout = pl.pallas_call(_wk, out_shape=wshape)(
        *w0s, *b0s, *w1s, *b1s, *w2s, *b2s,
        ws_w, ws_b[:, None], fc3_w, fc3_b[:, None], fc2_w,
        fc2_b.reshape(1, 1), fc1_w, fc1_b.reshape(1, 1),
        gat_b.reshape(8, T), edge_index.astype(jnp.int32))

    Ts = [wout[j].reshape(64, 1024) for j in range(3)]
    bNs = [wout[3 + j].reshape(1, 1024) for j in range(3)]
    Gps = [wout[6 + j].reshape(512, 256) for j in range(3)]
    Gss = [wout[9 + j].reshape(8, 256) for j in range(3)]
    bPs = [wout[12 + j].reshape(1, 256) for j in range(3)]
    F = wout[15].reshape(256, 512)
    bF = wout[16].reshape(1, 512)
    W2m = wout[17].reshape(512, 64)
    constv = wout[18]
    Mcnt = wout[19]

    xl3, xr3, pos_out = pl.pallas_call(
        _front_body,
        out_shape=[jax.ShapeDtypeStruct((B, N, H * C), _f32),
                   jax.ShapeDtypeStruct((B, N, H * C), _f32),
                   jax.ShapeDtypeStruct((R, 32 * POS), _f32)],
    )(s, p, *Ts, *bNs, *Gps, *Gss, *bPs, F, bF,
      gat_wl, gat_bl[None, :], gat_wr, gat_br[None, :])

    full2 = lambda b: (0, 0)
    pre = pl.pallas_call(
        _gat_body,
        grid=(B,),
        in_specs=[pl.BlockSpec((1, N, H * C), lambda b: (b, 0, 0)),
                  pl.BlockSpec((1, N, H * C), lambda b: (b, 0, 0)),
                  pl.BlockSpec((H, C), full2),
                  pl.BlockSpec((N, N), full2),
                  pl.BlockSpec((1, N), full2),
                  pl.BlockSpec((8 * T, T), full2),
                  pl.BlockSpec((1, T), full2)],
        out_specs=pl.BlockSpec((1, 1, T), lambda b: (b, 0, 0)),
        out_shape=jax.ShapeDtypeStruct((B, 1, T), _f32),
    )(xl3, xr3, gat_att, Mcnt, fc1_w, W2m, constv)

    pos_learned = jnp.transpose(pos_out.reshape(B, N, 32, POS), (0, 2, 1, 3))
    return (pre, target, pos_learned)


# speedup vs baseline: 1.0457x; 1.0457x over previous
import jax
import jax.numpy as jnp
from jax import lax
from jax.experimental import pallas as pl
from jax.experimental.pallas import tpu as pltpu

B, T, N, H, C = 16, 64, 62, 2, 256
POS = 16
E = 4000
R = B * N

_f32 = jnp.float32


def _iota(shape, dim):
    return lax.broadcasted_iota(jnp.int32, shape, dim)


def _weights_body(w0_refs, b0_refs, w1_refs, b1_refs, w2_refs, b2_refs,
                  wsw_ref, wsb_ref, fc3w_ref, fc3b_ref, fc2w_ref, fc2b_ref,
                  fc1w_ref, fc1b_ref, gatb8_ref, edge_ref,
                  T_refs, bN_refs, Gp_refs, Gs_refs, bP_refs,
                  F_ref, bF_ref, W2m_ref, cv_ref, M_ref):
    def sel(n_in, k):
        return ((_iota((2 * n_in, n_in), 0) == _iota((2 * n_in, n_in), 1) * 2 + k)
                ).astype(_f32)

    SEL32 = [sel(32, 0), sel(32, 1)]
    SEL16 = [sel(16, 0), sel(16, 1)]

    wsw = wsw_ref[...]
    wsb = wsb_ref[...]
    d3 = _iota((64, 1, 64), 2) - _iota((64, 1, 64), 0)
    d4 = _iota((1, 16, 1, 16), 3) - _iota((1, 16, 1, 16), 1)
    wrow = _iota((1, 80), 1)

    def dotf(a, b, dn):
        return lax.dot_general(a, b, (dn, ((), ())),
                               preferred_element_type=_f32)

    for j in range(3):
        w0 = w0_refs[j][...]
        w1 = w1_refs[j][...]
        w2 = w2_refs[j][...]
        L1 = [jnp.dot(w0, SEL32[1], preferred_element_type=_f32),
              jnp.dot(w0, SEL32[0], preferred_element_type=_f32)]
        L2 = [jnp.dot(w1, SEL16[1], preferred_element_type=_f32),
              jnp.dot(w1, SEL16[0], preferred_element_type=_f32)]
        L3 = [jnp.dot(w2, SEL16[1], preferred_element_type=_f32),
              jnp.dot(w2, SEL16[0], preferred_element_type=_f32)]

        Tacc = jnp.zeros((64, 16, 64), _f32)
        Gp = jnp.zeros((32, 16, 16, 16), _f32)
        Gs = jnp.zeros((8, 16, 16), _f32)
        betaT = (b2_refs[j][...]
                 * jnp.ones((1, 80), _f32))
        for a in (0, 1):
            for b in (0, 1):
                colab = jnp.dot(L3[a], jnp.dot(L2[b], b0_refs[j][...],
                                               preferred_element_type=_f32),
                                preferred_element_type=_f32)
                betaT = betaT + colab * (wrow >= 4 * a + 2 * b).astype(_f32)
            cola = jnp.dot(L3[a], b1_refs[j][...],
                           preferred_element_type=_f32)
            betaT = betaT + cola * (wrow >= 4 * a).astype(_f32)

        for t in range(8):
            a, b, c = (t >> 2) & 1, (t >> 1) & 1, t & 1
            P21 = jnp.dot(L2[b], L1[c], preferred_element_type=_f32)
            A_t = jnp.dot(L3[a], P21, preferred_element_type=_f32)
            AT_t = dotf(P21, L3[a], ((0,), (1,)))
            v_t = jnp.dot(A_t, wsw, preferred_element_type=_f32)
            awsb = jnp.dot(A_t, wsb, preferred_element_type=_f32)
            betaT = betaT + awsb * ((wrow >= t) & (wrow < 64 + t)).astype(_f32)
            Tacc = Tacc + (d3 == t).astype(_f32) * v_t[None, :, :]
            Gp = Gp + (d4 == t).astype(_f32) * AT_t[:, None, :, None]
            if t >= 1:
                m3 = (_iota((8, 1, 16), 2) == _iota((8, 1, 16), 0) - 8 + t)
                Gs = Gs + m3.astype(_f32) * v_t[None, :, :]

        T_refs[j][...] = Tacc
        bN_refs[j][...] = betaT[:, :64]
        bP_refs[j][...] = betaT[:, 64:]
        Gp_refs[j][...] = Gp
        Gs_refs[j][...] = Gs

    fc3T = jnp.transpose(fc3w_ref[...])
    mwp = (_iota((1, 16, 1, 16), 1) == _iota((1, 16, 1, 16), 3)).astype(_f32)
    F_ref[...] = mwp * fc3T[:, None, :, None]
    bF_ref[...] = fc3b_ref[...] * jnp.ones((1, 16), _f32)

    fc2T = jnp.transpose(fc2w_ref[...])
    mtv = (_iota((1, 64, 64), 1) == _iota((1, 64, 64), 2)).astype(_f32)
    W2m_ref[...] = mtv * fc2T[:, :, None]
    gbrow = jnp.sum(gatb8_ref[...] * fc2T, axis=0, keepdims=True)
    w1s = jnp.sum(fc1w_ref[...], axis=1, keepdims=True)
    cv_ref[...] = fc1b_ref[...] + w1s * fc2b_ref[...] + w1s * gbrow

    srcrow = edge_ref[0:1, :]
    dstrow = edge_ref[1:2, :]
    D = (_iota((N, E), 0) == dstrow).astype(_f32)
    S = (_iota((N, E), 0) == srcrow).astype(_f32)
    eye = (_iota((N, N), 0) == _iota((N, N), 1)).astype(_f32)
    M_ref[...] = dotf(D, S, ((1,), (1,))) + eye


def _front_body(s_ref, p_ref, T1_ref, T2_ref, T3_ref, bN1_ref, bN2_ref,
                bN3_ref, Gp1_ref, Gp2_ref, Gp3_ref, Gs1_ref, Gs2_ref, Gs3_ref,
                bP1_ref, bP2_ref, bP3_ref, F_ref, bF_ref,
                wl_ref, bl_ref, wr_ref, br_ref,
                xl_ref, xr_ref, pos_ref):
    s = s_ref[...]
    stail = s[:, 56:64]
    p = p_ref[...]

    def node_branch(T_r, bN_r):
        return jnp.dot(s, T_r[...], preferred_element_type=_f32) + bN_r[...]

    def pos_branch(Gp_r, Gs_r, bP_r):
        return (jnp.dot(p, Gp_r[...], preferred_element_type=_f32)
                + jnp.dot(stail, Gs_r[...], preferred_element_type=_f32)
                + bP_r[...])

    def glu(o1, o2, o3):
        return jnp.maximum(jnp.tanh(o1) * jax.nn.sigmoid(o2) + o3, 0.0)

    g_node = glu(node_branch(T1_ref, bN1_ref),
                 node_branch(T2_ref, bN2_ref),
                 node_branch(T3_ref, bN3_ref))
    g_pos = glu(pos_branch(Gp1_ref, Gs1_ref, bP1_ref),
                pos_branch(Gp2_ref, Gs2_ref, bP2_ref),
                pos_branch(Gp3_ref, Gs3_ref, bP3_ref))

    xl_ref[...] = (jnp.dot(g_node, wl_ref[...], preferred_element_type=_f32)
                   + bl_ref[...])
    xr_ref[...] = (jnp.dot(g_node, wr_ref[...], preferred_element_type=_f32)
                   + br_ref[...])
    pos_ref[...] = jnp.dot(g_pos, F_ref[...],
                           preferred_element_type=_f32) + bF_ref[...]


def _gat_body(xl_ref, xr_ref, att_ref, M_ref, w1_ref, W2m_ref, cv_ref,
              pre_ref):
    M = M_ref[...]
    mask = M > 0.0
    Xl_all = xl_ref[0]
    Xr_all = xr_ref[0]
    qs = []
    for h in range(H):
        Xl = Xl_all[:, h * C:(h + 1) * C]
        Xr = Xr_all[:, h * C:(h + 1) * C]
        att3 = att_ref[...][h][None, None, :]
        Z = Xl[None, :, :] + Xr[:, None, :]
        Z = jnp.where(Z >= 0.0, Z, 0.2 * Z)
        S = jnp.sum(Z * att3, axis=-1)
        Sm = jnp.where(mask, S, -1e30)
        mx = jnp.max(Sm, axis=1, keepdims=True)
        P = jnp.exp(Sm - mx) * M
        den = jnp.sum(P, axis=1, keepdims=True)
        Aw = P / (den + 1e-16)
        tau = jnp.dot(w1_ref[...], Aw, preferred_element_type=_f32)
        qs.append(jnp.dot(tau, Xl, preferred_element_type=_f32))
    q = jnp.concatenate(qs, axis=1)
    acc = jnp.dot(q, W2m_ref[...], preferred_element_type=_f32)
    pre_ref[...] = jax.nn.sigmoid(acc + cv_ref[...])[None]


def kernel(data, position_embed, edge_index, ws_w, ws_b,
           d1w0, d1b0, d1w1, d1b1, d1w2, d1b2,
           d2w0, d2b0, d2w1, d2b1, d2w2, d2b2,
           d3w0, d3b0, d3w1, d3b1, d3w2, d3b2,
           fc3_w, fc3_b, gat_wl, gat_bl, gat_wr, gat_br, gat_att, gat_b,
           fc2_w, fc2_b, fc1_w, fc1_b):
    data = data.astype(_f32)
    train = data[:, :62, :]
    target = data[:, 62:63, :]
    s = train.reshape(R, T)
    p = jnp.transpose(position_embed, (0, 2, 1, 3)).reshape(R, 32 * POS)

    w0s = [d1w0.reshape(16, 64), d2w0.reshape(16, 64), d3w0.reshape(16, 64)]
    w1s = [d1w1.reshape(16, 32), d2w1.reshape(16, 32), d3w1.reshape(16, 32)]
    w2s = [d1w2.reshape(16, 32), d2w2.reshape(16, 32), d3w2.reshape(16, 32)]
    b0s = [d1b0[:, None], d2b0[:, None], d3b0[:, None]]
    b1s = [d1b1[:, None], d2b1[:, None], d3b1[:, None]]
    b2s = [d1b2[:, None], d2b2[:, None], d3b2[:, None]]

    wshape = [jax.ShapeDtypeStruct((64, 16, 64), _f32)] * 3 \
        + [jax.ShapeDtypeStruct((16, 64), _f32)] * 3 \
        + [jax.ShapeDtypeStruct((32, 16, 16, 16), _f32)] * 3 \
        + [jax.ShapeDtypeStruct((8, 16, 16), _f32)] * 3 \
        + [jax.ShapeDtypeStruct((16, 16), _f32)] * 3 \
        + [jax.ShapeDtypeStruct((16, 16, 32, 16), _f32),
           jax.ShapeDtypeStruct((32, 16), _f32),
           jax.ShapeDtypeStruct((8, 64, 64), _f32),
           jax.ShapeDtypeStruct((1, 64), _f32),
           jax.ShapeDtypeStruct((N, N), _f32)]

    def _wk(*refs):
        _weights_body(refs[0:3], refs[3:6], refs[6:9], refs[9:12],
                      refs[12:15], refs[15:18], refs[18], refs[19], refs[20],
                      refs[21], refs[22], refs[23], refs[24], refs[25],
                      refs[26], refs[27],
                      refs[28:31], refs[31:34], refs[34:37], refs[37:40],
                      refs[40:43], refs[43], refs[44], refs[45], refs[46],
                      refs[47])

    wout = pl.pallas_call(_wk, out_shape=wshape)(
        *w0s, *b0s, *w1s, *b1s, *w2s, *b2s,
        ws_w, ws_b[:, None], fc3_w, fc3_b[:, None], fc2_w,
        fc2_b.reshape(1, 1), fc1_w, fc1_b.reshape(1, 1),
        gat_b.reshape(8, T), edge_index.astype(jnp.int32))

    Ts = [wout[j].reshape(64, 1024) for j in range(3)]
    bNs = [wout[3 + j].reshape(1, 1024) for j in range(3)]
    Gps = [wout[6 + j].reshape(512, 256) for j in range(3)]
    Gss = [wout[9 + j].reshape(8, 256) for j in range(3)]
    bPs = [wout[12 + j].reshape(1, 256) for j in range(3)]
    F = wout[15].reshape(256, 512)
    bF = wout[16].reshape(1, 512)
    W2m = wout[17].reshape(512, 64)
    constv = wout[18]
    Mcnt = wout[19]

    xl2, xr2, pos_out = pl.pallas_call(
        _front_body,
        out_shape=[jax.ShapeDtypeStruct((R, H * C), _f32),
                   jax.ShapeDtypeStruct((R, H * C), _f32),
                   jax.ShapeDtypeStruct((R, 32 * POS), _f32)],
    )(s, p, *Ts, *bNs, *Gps, *Gss, *bPs, F, bF,
      gat_wl, gat_bl[None, :], gat_wr, gat_br[None, :])
    xl3 = xl2.reshape(B, N, H * C)
    xr3 = xr2.reshape(B, N, H * C)

    full2 = lambda b: (0, 0)
    pre = pl.pallas_call(
        _gat_body,
        grid=(B,),
        in_specs=[pl.BlockSpec((1, N, H * C), lambda b: (b, 0, 0)),
                  pl.BlockSpec((1, N, H * C), lambda b: (b, 0, 0)),
                  pl.BlockSpec((H, C), full2),
                  pl.BlockSpec((N, N), full2),
                  pl.BlockSpec((1, N), full2),
                  pl.BlockSpec((8 * T, T), full2),
                  pl.BlockSpec((1, T), full2)],
        out_specs=pl.BlockSpec((1, 1, T), lambda b: (b, 0, 0)),
        out_shape=jax.ShapeDtypeStruct((B, 1, T), _f32),
    )(xl3, xr3, gat_att, Mcnt, fc1_w, W2m, constv)

    pos_learned = jnp.transpose(pos_out.reshape(B, N, 32, POS), (0, 2, 1, 3))
    return (pre, target, pos_learned)


# speedup vs baseline: 1.4208x; 1.3587x over previous
import jax
import jax.numpy as jnp
from jax import lax
from jax.experimental import pallas as pl
from jax.experimental.pallas import tpu as pltpu

B, T, N, H, C = 16, 64, 62, 2, 256
POS = 16
E = 4000
R = B * N

_f32 = jnp.float32


def _iota(shape, dim):
    return lax.broadcasted_iota(jnp.int32, shape, dim)


def _weights_body(w0_refs, b0_refs, w1_refs, b1_refs, w2_refs, b2_refs,
                  wsw_ref, wsb_ref, fc3w_ref, fc3b_ref, fc2w_ref, fc2b_ref,
                  fc1w_ref, fc1b_ref, gatb8_ref, edge_ref,
                  T_refs, bN_refs, Gp_refs, Gs_refs, bP_refs,
                  F_ref, bF_ref, W2m_ref, cv_ref, M_ref):
    def dotf(a, b, dn=((1,), (0,))):
        return lax.dot_general(a, b, (dn, ((), ())),
                               preferred_element_type=_f32)

    def rep_rows(nr, nb):
        return (_iota((nr, nb), 0) // (nr // nb) == _iota((nr, nb), 1)
                ).astype(_f32)

    def rep_cols(nb, nc):
        return (_iota((nb, nc), 0) == _iota((nb, nc), 1) // (nc // nb)
                ).astype(_f32)

    SEL32 = [((_iota((64, 32), 0) == _iota((64, 32), 1) * 2 + k)).astype(_f32)
             for k in (0, 1)]
    SEL16 = [((_iota((32, 16), 0) == _iota((32, 16), 1) * 2 + k)).astype(_f32)
             for k in (0, 1)]

    wsw = wsw_ref[...]
    wsb = wsb_ref[...]
    wrow = _iota((1, 80), 1)

    dT = _iota((64, 1024), 1) % 64 - _iota((64, 1024), 0)
    dG = _iota((512, 256), 1) % 16 - _iota((512, 256), 0) % 16
    wpGs = _iota((8, 256), 1) % 16
    mGs = _iota((8, 256), 0)
    REPv64 = rep_cols(16, 1024)
    REPv16 = rep_cols(16, 256)
    REPr512 = rep_rows(512, 32)
    REPc256 = rep_cols(16, 256)

    for j in range(3):
        w0 = w0_refs[j][...]
        w1 = w1_refs[j][...]
        w2 = w2_refs[j][...]
        L1 = [dotf(w0, SEL32[1]), dotf(w0, SEL32[0])]
        L2 = [dotf(w1, SEL16[1]), dotf(w1, SEL16[0])]
        L3 = [dotf(w2, SEL16[1]), dotf(w2, SEL16[0])]
        L3s = jnp.concatenate(L3, axis=0)
        L1c = jnp.concatenate(L1, axis=1)
        P21f = jnp.concatenate([dotf(L2[0], L1c), dotf(L2[1], L1c)], axis=1)
        A_all = dotf(L3s, P21f)

        Tacc = jnp.zeros((64, 1024), _f32)
        Gp = jnp.zeros((512, 256), _f32)
        Gs = jnp.zeros((8, 256), _f32)
        betaT = b2_refs[j][...] * jnp.ones((1, 80), _f32)
        L32 = dotf(L3s, jnp.concatenate(L2, axis=1))
        for a in (0, 1):
            for b in (0, 1):
                colab = dotf(L32[a * 16:(a + 1) * 16, b * 16:(b + 1) * 16],
                             b0_refs[j][...])
                betaT = betaT + colab * (wrow >= 4 * a + 2 * b).astype(_f32)
            cola = dotf(L3[a], b1_refs[j][...])
            betaT = betaT + cola * (wrow >= 4 * a).astype(_f32)

        for t in range(8):
            a, bc = (t >> 2) & 1, t & 3
            A_t = A_all[a * 16:(a + 1) * 16, bc * 32:(bc + 1) * 32]
            v_row = dotf(wsw, A_t, ((0,), (1,)))
            awsb = dotf(A_t, wsb)
            betaT = betaT + awsb * ((wrow >= t) & (wrow < 64 + t)).astype(_f32)
            Tacc = Tacc + (dT == t).astype(_f32) * dotf(v_row, REPv64)
            ATx = dotf(REPr512, A_t, ((1,), (1,)))
            Gp = Gp + (dG == t).astype(_f32) * dotf(ATx, REPc256)
            if t >= 1:
                Gs = Gs + ((wpGs == mGs - 8 + t).astype(_f32)
                           * dotf(v_row, REPv16))

        T_refs[j][...] = Tacc
        bN_refs[j][...] = betaT[:, :64]
        bP_refs[j][...] = betaT[:, 64:]
        Gp_refs[j][...] = Gp
        Gs_refs[j][...] = Gs

    mF = (_iota((256, 512), 0) % 16 == _iota((256, 512), 1) % 16).astype(_f32)
    fc3x = dotf(dotf(rep_rows(256, 16), fc3w_ref[...], ((1,), (1,))),
                rep_cols(32, 512))
    F_ref[...] = mF * fc3x
    bF_ref[...] = dotf(fc3b_ref[...], rep_cols(32, 512), ((0,), (0,)))

    mW2 = (_iota((512, 64), 0) % 64 == _iota((512, 64), 1)).astype(_f32)
    fc2x = dotf(rep_rows(512, 8), fc2w_ref[...], ((1,), (1,)))
    W2m_ref[...] = mW2 * fc2x
    gbrow = dotf(fc2w_ref[...], gatb8_ref[...])
    w1s = jnp.sum(fc1w_ref[...], axis=1, keepdims=True)
    cv_ref[...] = fc1b_ref[...] + w1s * fc2b_ref[...] + w1s * gbrow

    srcrow = edge_ref[0:1, :]
    dstrow = edge_ref[1:2, :]
    D = (_iota((N, E), 0) == dstrow).astype(_f32)
    S = (_iota((N, E), 0) == srcrow).astype(_f32)
    eye = (_iota((N, N), 0) == _iota((N, N), 1)).astype(_f32)
    M_ref[...] = dotf(D, S, ((1,), (1,))) + eye


def _front_body(s_ref, p_ref, T1_ref, T2_ref, T3_ref, bN1_ref, bN2_ref,
                bN3_ref, Gp1_ref, Gp2_ref, Gp3_ref, Gs1_ref, Gs2_ref, Gs3_ref,
                bP1_ref, bP2_ref, bP3_ref, F_ref, bF_ref,
                wl_ref, bl_ref, wr_ref, br_ref,
                xl_ref, xr_ref, pos_ref):
    s = s_ref[...]
    stail = s[:, 56:64]
    p = p_ref[...]

    def node_branch(T_r, bN_r):
        return jnp.dot(s, T_r[...], preferred_element_type=_f32) + bN_r[...]

    def pos_branch(Gp_r, Gs_r, bP_r):
        return (jnp.dot(p, Gp_r[...], preferred_element_type=_f32)
                + jnp.dot(stail, Gs_r[...], preferred_element_type=_f32)
                + bP_r[...])

    def glu(o1, o2, o3):
        return jnp.maximum(jnp.tanh(o1) * jax.nn.sigmoid(o2) + o3, 0.0)

    g_node = glu(node_branch(T1_ref, bN1_ref),
                 node_branch(T2_ref, bN2_ref),
                 node_branch(T3_ref, bN3_ref))
    g_pos = glu(pos_branch(Gp1_ref, Gs1_ref, bP1_ref),
                pos_branch(Gp2_ref, Gs2_ref, bP2_ref),
                pos_branch(Gp3_ref, Gs3_ref, bP3_ref))

    xl_ref[...] = (jnp.dot(g_node, wl_ref[...], preferred_element_type=_f32)
                   + bl_ref[...])
    xr_ref[...] = (jnp.dot(g_node, wr_ref[...], preferred_element_type=_f32)
                   + br_ref[...])
    pos_ref[...] = jnp.dot(g_pos, F_ref[...],
                           preferred_element_type=_f32) + bF_ref[...]


def _gat_body(xl_ref, xr_ref, att_ref, M_ref, w1_ref, W2m_ref, cv_ref,
              pre_ref):
    M = M_ref[...]
    mask = M > 0.0
    Xl_all = xl_ref[0]
    Xr_all = xr_ref[0]
    qs = []
    for h in range(H):
        Xl = Xl_all[:, h * C:(h + 1) * C]
        Xr = Xr_all[:, h * C:(h + 1) * C]
        att3 = att_ref[...][h][None, None, :]
        Z = Xl[None, :, :] + Xr[:, None, :]
        Z = jnp.where(Z >= 0.0, Z, 0.2 * Z)
        S = jnp.sum(Z * att3, axis=-1)
        Sm = jnp.where(mask, S, -1e30)
        mx = jnp.max(Sm, axis=1, keepdims=True)
        P = jnp.exp(Sm - mx) * M
        den = jnp.sum(P, axis=1, keepdims=True)
        Aw = P / (den + 1e-16)
        tau = jnp.dot(w1_ref[...], Aw, preferred_element_type=_f32)
        qs.append(jnp.dot(tau, Xl, preferred_element_type=_f32))
    q = jnp.concatenate(qs, axis=1)
    acc = jnp.dot(q, W2m_ref[...], preferred_element_type=_f32)
    pre_ref[...] = jax.nn.sigmoid(acc + cv_ref[...])[None]


def kernel(data, position_embed, edge_index, ws_w, ws_b,
           d1w0, d1b0, d1w1, d1b1, d1w2, d1b2,
           d2w0, d2b0, d2w1, d2b1, d2w2, d2b2,
           d3w0, d3b0, d3w1, d3b1, d3w2, d3b2,
           fc3_w, fc3_b, gat_wl, gat_bl, gat_wr, gat_br, gat_att, gat_b,
           fc2_w, fc2_b, fc1_w, fc1_b):
    data = data.astype(_f32)
    train = data[:, :62, :]
    target = data[:, 62:63, :]
    s = train.reshape(R, T)
    p = jnp.transpose(position_embed, (0, 2, 1, 3)).reshape(R, 32 * POS)

    w0s = [d1w0.reshape(16, 64), d2w0.reshape(16, 64), d3w0.reshape(16, 64)]
    w1s = [d1w1.reshape(16, 32), d2w1.reshape(16, 32), d3w1.reshape(16, 32)]
    w2s = [d1w2.reshape(16, 32), d2w2.reshape(16, 32), d3w2.reshape(16, 32)]
    b0s = [d1b0[:, None], d2b0[:, None], d3b0[:, None]]
    b1s = [d1b1[:, None], d2b1[:, None], d3b1[:, None]]
    b2s = [d1b2[:, None], d2b2[:, None], d3b2[:, None]]

    wshape = [jax.ShapeDtypeStruct((64, 1024), _f32)] * 3 \
        + [jax.ShapeDtypeStruct((16, 64), _f32)] * 3 \
        + [jax.ShapeDtypeStruct((512, 256), _f32)] * 3 \
        + [jax.ShapeDtypeStruct((8, 256), _f32)] * 3 \
        + [jax.ShapeDtypeStruct((16, 16), _f32)] * 3 \
        + [jax.ShapeDtypeStruct((256, 512), _f32),
           jax.ShapeDtypeStruct((1, 512), _f32),
           jax.ShapeDtypeStruct((512, 64), _f32),
           jax.ShapeDtypeStruct((1, 64), _f32),
           jax.ShapeDtypeStruct((N, N), _f32)]

    def _wk(*refs):
        _weights_body(refs[0:3], refs[3:6], refs[6:9], refs[9:12],
                      refs[12:15], refs[15:18], refs[18], refs[19], refs[20],
                      refs[21], refs[22], refs[23], refs[24], refs[25],
                      refs[26], refs[27],
                      refs[28:31], refs[31:34], refs[34:37], refs[37:40],
                      refs[40:43], refs[43], refs[44], refs[45], refs[46],
                      refs[47])

    wout = pl.pallas_call(_wk, out_shape=wshape)(
        *w0s, *b0s, *w1s, *b1s, *w2s, *b2s,
        ws_w, ws_b[:, None], fc3_w, fc3_b[:, None], fc2_w,
        fc2_b.reshape(1, 1), fc1_w, fc1_b.reshape(1, 1),
        gat_b.reshape(8, T), edge_index.astype(jnp.int32))

    Ts = [wout[j] for j in range(3)]
    bNs = [wout[3 + j].reshape(1, 1024) for j in range(3)]
    Gps = [wout[6 + j] for j in range(3)]
    Gss = [wout[9 + j] for j in range(3)]
    bPs = [wout[12 + j].reshape(1, 256) for j in range(3)]
    F = wout[15]
    bF = wout[16]
    W2m = wout[17]
    constv = wout[18]
    Mcnt = wout[19]

    xl2, xr2, pos_out = pl.pallas_call(
        _front_body,
        out_shape=[jax.ShapeDtypeStruct((R, H * C), _f32),
                   jax.ShapeDtypeStruct((R, H * C), _f32),
                   jax.ShapeDtypeStruct((R, 32 * POS), _f32)],
    )(s, p, *Ts, *bNs, *Gps, *Gss, *bPs, F, bF,
      gat_wl, gat_bl[None, :], gat_wr, gat_br[None, :])
    xl3 = xl2.reshape(B, N, H * C)
    xr3 = xr2.reshape(B, N, H * C)

    full2 = lambda b: (0, 0)
    pre = pl.pallas_call(
        _gat_body,
        grid=(B,),
        in_specs=[pl.BlockSpec((1, N, H * C), lambda b: (b, 0, 0)),
                  pl.BlockSpec((1, N, H * C), lambda b: (b, 0, 0)),
                  pl.BlockSpec((H, C), full2),
                  pl.BlockSpec((N, N), full2),
                  pl.BlockSpec((1, N), full2),
                  pl.BlockSpec((8 * T, T), full2),
                  pl.BlockSpec((1, T), full2)],
        out_specs=pl.BlockSpec((1, 1, T), lambda b: (b, 0, 0)),
        out_shape=jax.ShapeDtypeStruct((B, 1, T), _f32),
    )(xl3, xr3, gat_att, Mcnt, fc1_w, W2m, constv)

    pos_learned = jnp.transpose(pos_out.reshape(B, N, 32, POS), (0, 2, 1, 3))
    return (pre, target, pos_learned)
